# baseline (speedup 1.0000x reference)
import functools
import math

import jax
import jax.numpy as jnp
from jax import lax
from jax.experimental import pallas as pl
from jax.experimental.pallas import tpu as pltpu
from jax.experimental.pallas import tpu_sc as plsc

H = 64
SCALE = float(math.sqrt(H))
NW = 32
CH = 256
GB = 128
NSUB = CH // GB
NBUF = 2


def _make_encoder(N):
    per_w = N // NW
    chunks = per_w // CH
    assert chunks % NBUF == 0
    mesh = plsc.VectorSubcoreMesh(core_axis_name="c", subcore_axis_name="s")

    @functools.partial(
        pl.kernel,
        mesh=mesh,
        compiler_params=pltpu.CompilerParams(use_tc_tiling_on_sc=False),
        out_type=jax.ShapeDtypeStruct((N, H), jnp.float32),
        scratch_types=[
            pltpu.VMEM((NBUF, NSUB, GB), jnp.int32),
            pltpu.VMEM((NBUF, NSUB, GB), jnp.int32),
            pltpu.VMEM((NBUF, CH, H), jnp.float32),
            pltpu.VMEM((NBUF, CH, H), jnp.float32),
            pltpu.VMEM((NBUF, CH), jnp.float32),
            pltpu.VMEM((H,), jnp.float32),
            pltpu.SemaphoreType.DMA((NBUF,)),
            pltpu.SemaphoreType.DMA((NBUF,)),
        ],
    )
    def enc(tok_hbm, typ_hbm, dur_hbm, emb_hbm, t2_hbm, dw_hbm,
            out_hbm, idx_v, tidx_v, rows_v, t2rows_v, dur_v, dw_v,
            gsem, osem):
        cid = lax.axis_index("c")
        sid = lax.axis_index("s")
        wid = sid * 2 + cid
        base = wid * per_w

        pltpu.sync_copy(dw_hbm, dw_v)
        dwv = [dw_v[pl.ds(j * 16, 16)] for j in range(4)]

        def issue(g, b):
            row0 = base + g * CH
            r128 = wid * (per_w // GB) + g * NSUB
            pltpu.sync_copy(tok_hbm.at[pl.ds(r128, NSUB)], idx_v.at[b])
            pltpu.sync_copy(typ_hbm.at[pl.ds(r128, NSUB)], tidx_v.at[b])
            for jb in range(NSUB):
                pltpu.async_copy(emb_hbm.at[idx_v.at[b].at[jb]],
                                 rows_v.at[b].at[pl.ds(jb * GB, GB)],
                                 gsem.at[b])
                pltpu.async_copy(t2_hbm.at[tidx_v.at[b].at[jb]],
                                 t2rows_v.at[b].at[pl.ds(jb * GB, GB)],
                                 gsem.at[b])
            pltpu.async_copy(dur_hbm.at[pl.ds(row0, CH)], dur_v.at[b],
                             gsem.at[b])

        def drain_gathers(b):
            for jb in range(NSUB):
                pltpu.make_async_copy(emb_hbm.at[pl.ds(0, GB)],
                                      rows_v.at[b].at[pl.ds(jb * GB, GB)],
                                      gsem.at[b]).wait()
                pltpu.make_async_copy(emb_hbm.at[pl.ds(0, GB)],
                                      t2rows_v.at[b].at[pl.ds(jb * GB, GB)],
                                      gsem.at[b]).wait()
            pltpu.make_async_copy(dur_hbm.at[pl.ds(0, CH)], dur_v.at[b],
                                  gsem.at[b]).wait()

        def wait_scatter(b):
            pltpu.make_async_copy(rows_v.at[b],
                                  out_hbm.at[pl.ds(base, CH)],
                                  osem.at[b]).wait()

        def compute(g, b):
            rb = rows_v.at[b]
            tb = t2rows_v.at[b]

            def grp_body(q, c2):
                dur16 = dur_v[b, pl.ds(q * 16, 16)]
                for k in range(16):
                    i = q * 16 + k
                    d16 = jnp.full((16,), dur16[k], dtype=jnp.float32)
                    for j in range(4):
                        sl = pl.ds(j * 16, 16)
                        t = tb[i, sl]
                        e = rb[i, sl]
                        rb[i, sl] = e * SCALE + (d16 * dwv[j] + t)
                return c2

            lax.fori_loop(0, CH // 16, grp_body, 0)
            row0 = base + g * CH
            pltpu.async_copy(rb, out_hbm.at[pl.ds(row0, CH)], osem.at[b])

        issue(0, 0)

        def pair_body(it, carry):
            g0 = it * 2

            @pl.when(it >= 1)
            def _():
                wait_scatter(1)

            issue(g0 + 1, 1)
            drain_gathers(0)
            compute(g0, 0)

            @pl.when(g0 + 2 < chunks)
            def _():
                wait_scatter(0)
                issue(g0 + 2, 0)

            drain_gathers(1)
            compute(g0 + 1, 1)
            return carry

        lax.fori_loop(0, chunks // 2, pair_body, 0)
        wait_scatter(0)
        wait_scatter(1)

    return enc


def kernel(note_tokens, note_durs, note_types, emb_weight, type_emb_weight,
           dur_w, dur_b):
    B, L = note_tokens.shape
    N = B * L
    enc = _make_encoder(N)
    tok = note_tokens.reshape(N // GB, GB).astype(jnp.int32)
    typ = note_types.reshape(N // GB, GB).astype(jnp.int32)
    dur = note_durs.reshape(N)
    t2 = type_emb_weight * SCALE + dur_b[None, :]
    out = enc(tok, typ, dur, emb_weight, t2, dur_w)
    return out.reshape(B, L, H)


# speedup vs baseline: 5.5799x; 5.5799x over previous
import functools
import math

import jax
import jax.numpy as jnp
from jax import lax
from jax.experimental import pallas as pl
from jax.experimental.pallas import tpu as pltpu
from jax.experimental.pallas import tpu_sc as plsc

H = 64
SCALE = float(math.sqrt(H))
NW = 32
CH = 256
GB = 128
NSUB = CH // GB
NBUF = 2


def _make_encoder(N):
    per_w = N // NW
    chunks = per_w // CH
    assert chunks % NBUF == 0
    mesh = plsc.VectorSubcoreMesh(core_axis_name="c", subcore_axis_name="s")

    @functools.partial(
        pl.kernel,
        mesh=mesh,
        compiler_params=pltpu.CompilerParams(use_tc_tiling_on_sc=False),
        out_type=jax.ShapeDtypeStruct((N, H), jnp.float32),
        scratch_types=[
            pltpu.VMEM((NBUF, NSUB, GB), jnp.int32),
            pltpu.VMEM((NBUF, CH, H), jnp.float32),
            pltpu.VMEM((NBUF, CH), jnp.int32),
            pltpu.VMEM((NBUF, CH), jnp.float32),
            pltpu.VMEM((5, H), jnp.float32),
            pltpu.VMEM((5 * H,), jnp.float32),
            pltpu.VMEM((H,), jnp.float32),
            pltpu.VMEM((H,), jnp.float32),
            pltpu.SemaphoreType.DMA((NBUF,)),
            pltpu.SemaphoreType.DMA((NBUF,)),
        ],
    )
    def enc(tok_hbm, typ_hbm, dur_hbm, emb_hbm, te_hbm, dw_hbm, db_hbm,
            out_hbm, idx_v, rows_v, typ_v, dur_v, te_v, t2_v, dw_v, db_v,
            gsem, osem):
        wid = lax.axis_index("s") * 2 + lax.axis_index("c")
        base = wid * per_w

        pltpu.sync_copy(dw_hbm, dw_v)
        pltpu.sync_copy(db_hbm, db_v)
        pltpu.sync_copy(te_hbm, te_v)
        for r in range(5):
            for j in range(4):
                sl = pl.ds(j * 16, 16)
                t2_v[pl.ds(r * H + j * 16, 16)] = te_v[r, sl] * SCALE + db_v[sl]

        dwv = [dw_v[pl.ds(j * 16, 16)] for j in range(4)]

        def issue(g, b):
            row0 = base + g * CH
            r128 = wid * (per_w // GB) + g * NSUB
            pltpu.sync_copy(tok_hbm.at[pl.ds(r128, NSUB)], idx_v.at[b])
            for jb in range(NSUB):
                pltpu.async_copy(emb_hbm.at[idx_v.at[b].at[jb]],
                                 rows_v.at[b].at[pl.ds(jb * GB, GB)],
                                 gsem.at[b])
            pltpu.async_copy(typ_hbm.at[pl.ds(row0, CH)], typ_v.at[b],
                             gsem.at[b])
            pltpu.async_copy(dur_hbm.at[pl.ds(row0, CH)], dur_v.at[b],
                             gsem.at[b])

        def drain_gathers(b):
            for jb in range(NSUB):
                pltpu.make_async_copy(emb_hbm.at[pl.ds(0, GB)],
                                      rows_v.at[b].at[pl.ds(jb * GB, GB)],
                                      gsem.at[b]).wait()
            pltpu.make_async_copy(typ_hbm.at[pl.ds(0, CH)], typ_v.at[b],
                                  gsem.at[b]).wait()
            pltpu.make_async_copy(dur_hbm.at[pl.ds(0, CH)], dur_v.at[b],
                                  gsem.at[b]).wait()

        def wait_scatter(b):
            pltpu.make_async_copy(rows_v.at[b],
                                  out_hbm.at[pl.ds(base, CH)],
                                  osem.at[b]).wait()

        def compute(g, b):
            rb = rows_v.at[b]

            def grp_body(q, c2):
                dur16 = dur_v[b, pl.ds(q * 16, 16)]
                typ16 = typ_v[b, pl.ds(q * 16, 16)]
                for k in range(16):
                    i = q * 16 + k
                    d16 = jnp.full((16,), dur16[k], dtype=jnp.float32)
                    toff = typ16[k] * H
                    for j in range(4):
                        sl = pl.ds(j * 16, 16)
                        t = t2_v[pl.ds(toff + j * 16, 16)]
                        e = rb[i, sl]
                        rb[i, sl] = e * SCALE + (d16 * dwv[j] + t)
                return c2

            lax.fori_loop(0, CH // 16, grp_body, 0)
            row0 = base + g * CH
            pltpu.async_copy(rb, out_hbm.at[pl.ds(row0, CH)], osem.at[b])

        issue(0, 0)

        def pair_body(it, carry):
            g0 = it * 2

            @pl.when(it >= 1)
            def _():
                wait_scatter(1)

            issue(g0 + 1, 1)
            drain_gathers(0)
            compute(g0, 0)

            @pl.when(g0 + 2 < chunks)
            def _():
                wait_scatter(0)
                issue(g0 + 2, 0)

            drain_gathers(1)
            compute(g0 + 1, 1)
            return carry

        lax.fori_loop(0, chunks // 2, pair_body, 0)
        wait_scatter(0)
        wait_scatter(1)

    return enc


def kernel(note_tokens, note_durs, note_types, emb_weight, type_emb_weight,
           dur_w, dur_b):
    B, L = note_tokens.shape
    N = B * L
    enc = _make_encoder(N)
    tok = note_tokens.reshape(N // GB, GB).astype(jnp.int32)
    typ = note_types.reshape(N).astype(jnp.int32)
    dur = note_durs.reshape(N)
    out = enc(tok, typ, dur, emb_weight, type_emb_weight, dur_w, dur_b)
    return out.reshape(B, L, H)
